# baseline (speedup 1.0000x reference)
import functools

import jax
import jax.numpy as jnp
from jax import lax
from jax.experimental import pallas as pl
from jax.experimental.pallas import tpu as pltpu
from jax.experimental.pallas import tpu_sc as plsc

_N_CHANNELS = 4096
_N_BINS = 8192
_LANES = 16

_TC_ROWS = 0
_SC_ROWS = _N_CHANNELS - _TC_ROWS

_ROW_BLOCK = 128
_NW = 32
_SC_ROWS_PER_W = _SC_ROWS // _NW if _SC_ROWS else 0
_RCHUNK = 4
_NBUF = 2


def _flux_body(params_ref, energies_ref, flux_ref):
    alpha = params_ref[0]
    norm = params_ref[1]
    oma = 1.0 - alpha
    e_low = energies_ref[0, :]
    e_high = energies_ref[1, :]
    flux_ref[0, :] = norm * (jnp.exp(oma * jnp.log(e_high))
                             - jnp.exp(oma * jnp.log(e_low))) / oma


def _compute_flux(parameters, energies):
    return pl.pallas_call(
        _flux_body,
        in_specs=[
            pl.BlockSpec(memory_space=pltpu.SMEM),
            pl.BlockSpec((2, _N_BINS), lambda: (0, 0)),
        ],
        out_specs=pl.BlockSpec((1, _N_BINS), lambda: (0, 0)),
        out_shape=jax.ShapeDtypeStruct((1, _N_BINS), jnp.float32),
    )(parameters, energies)


def _tc_mv_body(flux_ref, m_ref, out_ref):
    m = m_ref[...]
    acc = jnp.sum(m * flux_ref[0, :][None, :], axis=1)
    out_ref[0, 0, :] = jnp.maximum(acc, 1e-6)


def _tc_matvec(flux, transfer_matrix):
    n_blocks = _TC_ROWS // _ROW_BLOCK
    out = pl.pallas_call(
        _tc_mv_body,
        grid=(n_blocks,),
        in_specs=[
            pl.BlockSpec((1, _N_BINS), lambda i: (0, 0)),
            pl.BlockSpec((_ROW_BLOCK, _N_BINS), lambda i: (i, 0)),
        ],
        out_specs=pl.BlockSpec((1, 1, _ROW_BLOCK), lambda i: (i, 0, 0)),
        out_shape=jax.ShapeDtypeStruct((n_blocks, 1, _ROW_BLOCK), jnp.float32),
    )(flux, transfer_matrix)
    return out.reshape(_TC_ROWS)


def _sc_mv_body(flux_hbm, tm_hbm, out_hbm, flux_v, ring, out_v,
                sem0, sem1):
    n_chunks = _SC_ROWS_PER_W // _RCHUNK
    group = _LANES // _RCHUNK
    wid = lax.axis_index("s") * 2 + lax.axis_index("c")
    row0 = _TC_ROWS + wid * _SC_ROWS_PER_W
    sems = (sem0, sem1)
    iota16 = lax.iota(jnp.int32, _LANES)

    pltpu.sync_copy(flux_hbm, flux_v)

    def chunk_copy(k, b):
        src = tm_hbm.at[pl.ds(row0 + k * _RCHUNK, _RCHUNK), :]
        return pltpu.make_async_copy(src, ring.at[b], sems[b])

    for b in range(_NBUF):
        chunk_copy(b, b).start()

    def do_chunk(k, b, g, res):
        chunk_copy(k, b).wait()
        zero = jnp.zeros((_LANES,), jnp.float32)

        def inner(j, accs):
            col = pl.multiple_of(j * _LANES, _LANES)
            f = flux_v[pl.ds(col, _LANES)]
            return tuple(acc + ring[b, i, pl.ds(col, _LANES)] * f
                         for i, acc in enumerate(accs))

        accs = lax.fori_loop(0, _N_BINS // _LANES, inner,
                             (zero,) * _RCHUNK)
        for i in range(_RCHUNK):
            s = plsc.cumsum(accs[i])[_LANES - 1]
            res = jnp.where(iota16 == g * _RCHUNK + i, s, res)

        @pl.when(k + _NBUF < n_chunks)
        def _():
            chunk_copy(k + _NBUF, b).start()
        return res

    def outer(grp, carry):
        k0 = grp * group
        res = jnp.zeros((_LANES,), jnp.float32)
        for j in range(group):
            res = do_chunk(k0 + j, j % _NBUF, j, res)
        base = pl.multiple_of(k0 * _RCHUNK, _LANES)
        out_v[pl.ds(base, _LANES)] = jnp.maximum(res, 1e-6)
        return carry

    lax.fori_loop(0, n_chunks // group, outer, 0)
    pltpu.sync_copy(out_v, out_hbm.at[pl.ds(wid * _SC_ROWS_PER_W,
                                            _SC_ROWS_PER_W)])


def _sc_matvec(flux, transfer_matrix):
    mesh = plsc.VectorSubcoreMesh(core_axis_name="c", subcore_axis_name="s")
    kern = functools.partial(
        pl.kernel,
        out_type=jax.ShapeDtypeStruct((_SC_ROWS,), jnp.float32),
        mesh=mesh,
        compiler_params=pltpu.CompilerParams(needs_layout_passes=False),
        scratch_types=[
            pltpu.VMEM((_N_BINS,), jnp.float32),
            pltpu.VMEM((_NBUF, _RCHUNK, _N_BINS), jnp.float32),
            pltpu.VMEM((_SC_ROWS_PER_W,), jnp.float32),
            pltpu.SemaphoreType.DMA,
            pltpu.SemaphoreType.DMA,
        ],
    )(_sc_mv_body)
    return kern(flux.reshape(_N_BINS), transfer_matrix)


def kernel(parameters, energies, transfer_matrix):
    flux = _compute_flux(parameters, energies)
    parts = []
    if _TC_ROWS:
        parts.append(_tc_matvec(flux, transfer_matrix))
    if _SC_ROWS:
        parts.append(_sc_matvec(flux, transfer_matrix))
    if len(parts) == 1:
        return parts[0]
    return jnp.concatenate(parts)


# speedup vs baseline: 1.3608x; 1.3608x over previous
import functools

import jax
import jax.numpy as jnp
from jax import lax
from jax.experimental import pallas as pl
from jax.experimental.pallas import tpu as pltpu
from jax.experimental.pallas import tpu_sc as plsc

_N_CHANNELS = 4096
_N_BINS = 8192
_LANES = 16

_TC_ROWS = 2560
_SC_ROWS = _N_CHANNELS - _TC_ROWS

_ROW_BLOCK = 128
_NW = 32
_SC_ROWS_PER_W = _SC_ROWS // _NW if _SC_ROWS else 0
_RCHUNK = 4
_NBUF = 2


def _flux_body(params_ref, energies_ref, flux_ref):
    alpha = params_ref[0]
    norm = params_ref[1]
    oma = 1.0 - alpha
    e_low = energies_ref[0, :]
    e_high = energies_ref[1, :]
    flux_ref[0, :] = norm * (jnp.exp(oma * jnp.log(e_high))
                             - jnp.exp(oma * jnp.log(e_low))) / oma


def _compute_flux(parameters, energies):
    return pl.pallas_call(
        _flux_body,
        in_specs=[
            pl.BlockSpec(memory_space=pltpu.SMEM),
            pl.BlockSpec((2, _N_BINS), lambda: (0, 0)),
        ],
        out_specs=pl.BlockSpec((1, _N_BINS), lambda: (0, 0)),
        out_shape=jax.ShapeDtypeStruct((1, _N_BINS), jnp.float32),
    )(parameters, energies)


def _tc_mv_body(flux_ref, m_ref, out_ref):
    m = m_ref[...]
    acc = jnp.sum(m * flux_ref[0, :][None, :], axis=1)
    out_ref[0, 0, :] = jnp.maximum(acc, 1e-6)


def _tc_matvec(flux, transfer_matrix):
    n_blocks = _TC_ROWS // _ROW_BLOCK
    out = pl.pallas_call(
        _tc_mv_body,
        grid=(n_blocks,),
        in_specs=[
            pl.BlockSpec((1, _N_BINS), lambda i: (0, 0)),
            pl.BlockSpec((_ROW_BLOCK, _N_BINS), lambda i: (i, 0)),
        ],
        out_specs=pl.BlockSpec((1, 1, _ROW_BLOCK), lambda i: (i, 0, 0)),
        out_shape=jax.ShapeDtypeStruct((n_blocks, 1, _ROW_BLOCK), jnp.float32),
    )(flux, transfer_matrix)
    return out.reshape(_TC_ROWS)


def _sc_mv_body(flux_hbm, tm_hbm, out_hbm, flux_v, ring, out_v,
                sem0, sem1):
    n_chunks = _SC_ROWS_PER_W // _RCHUNK
    group = _LANES // _RCHUNK
    wid = lax.axis_index("s") * 2 + lax.axis_index("c")
    row0 = _TC_ROWS + wid * _SC_ROWS_PER_W
    sems = (sem0, sem1)
    iota16 = lax.iota(jnp.int32, _LANES)

    pltpu.sync_copy(flux_hbm, flux_v)

    def chunk_copy(k, b):
        src = tm_hbm.at[pl.ds(row0 + k * _RCHUNK, _RCHUNK), :]
        return pltpu.make_async_copy(src, ring.at[b], sems[b])

    for b in range(_NBUF):
        chunk_copy(b, b).start()

    def do_chunk(k, b, g, res):
        chunk_copy(k, b).wait()
        zero = jnp.zeros((_LANES,), jnp.float32)

        def inner(j, accs):
            col = pl.multiple_of(j * _LANES, _LANES)
            f = flux_v[pl.ds(col, _LANES)]
            return tuple(acc + ring[b, i, pl.ds(col, _LANES)] * f
                         for i, acc in enumerate(accs))

        accs = lax.fori_loop(0, _N_BINS // _LANES, inner,
                             (zero,) * _RCHUNK)
        for i in range(_RCHUNK):
            s = plsc.cumsum(accs[i])[_LANES - 1]
            res = jnp.where(iota16 == g * _RCHUNK + i, s, res)

        @pl.when(k + _NBUF < n_chunks)
        def _():
            chunk_copy(k + _NBUF, b).start()
        return res

    def outer(grp, carry):
        k0 = grp * group
        res = jnp.zeros((_LANES,), jnp.float32)
        for j in range(group):
            res = do_chunk(k0 + j, j % _NBUF, j, res)
        base = pl.multiple_of(k0 * _RCHUNK, _LANES)
        out_v[pl.ds(base, _LANES)] = jnp.maximum(res, 1e-6)
        return carry

    lax.fori_loop(0, n_chunks // group, outer, 0)
    pltpu.sync_copy(out_v, out_hbm.at[pl.ds(wid * _SC_ROWS_PER_W,
                                            _SC_ROWS_PER_W)])


def _sc_matvec(flux, transfer_matrix):
    mesh = plsc.VectorSubcoreMesh(core_axis_name="c", subcore_axis_name="s")
    kern = functools.partial(
        pl.kernel,
        out_type=jax.ShapeDtypeStruct((_SC_ROWS,), jnp.float32),
        mesh=mesh,
        compiler_params=pltpu.CompilerParams(needs_layout_passes=False),
        scratch_types=[
            pltpu.VMEM((_N_BINS,), jnp.float32),
            pltpu.VMEM((_NBUF, _RCHUNK, _N_BINS), jnp.float32),
            pltpu.VMEM((_SC_ROWS_PER_W,), jnp.float32),
            pltpu.SemaphoreType.DMA,
            pltpu.SemaphoreType.DMA,
        ],
    )(_sc_mv_body)
    return kern(flux.reshape(_N_BINS), transfer_matrix)


def kernel(parameters, energies, transfer_matrix):
    flux = _compute_flux(parameters, energies)
    parts = []
    if _TC_ROWS:
        parts.append(_tc_matvec(flux, transfer_matrix))
    if _SC_ROWS:
        parts.append(_sc_matvec(flux, transfer_matrix))
    if len(parts) == 1:
        return parts[0]
    return jnp.concatenate(parts)
